# baseline (speedup 1.0000x reference)
import numpy as np
from math import comb, factorial, floor

import jax
import jax.numpy as jnp
from jax.experimental import pallas as pl

_MAX_L = 4


def _tables(max_l):
    dst, pows, cs, ns = [], [], [], []
    for l in range(max_l + 1):
        for m in range(-l, l + 1):
            am = abs(m)
            n_lm = (1.0 / (2.0 ** am * factorial(l))) * np.sqrt(
                2.0 * factorial(l + am) * factorial(l - am) / (2.0 if m == 0 else 1.0))
            ns.append(n_lm)
            vm = 0.5 if m < 0 else 0.0
            vmax = floor(am / 2.0 - vm) + vm
            for t in range(0, (l - am) // 2 + 1):
                for u in range(0, t + 1):
                    v = vm
                    while v <= vmax + 1e-9:
                        c = ((-1.0) ** int(round(t + v - vm))) * (0.25 ** t) \
                            * comb(l, t) * comb(l - t, am + t) * comb(t, u) * comb(am, int(round(2 * v)))
                        dst.append(l * (l + 1) + m)
                        pows.append([int(round(2 * t + am - 2 * (u + v))),
                                     int(round(2 * (u + v))),
                                     int(l - 2 * t - am)])
                        cs.append(c)
                        v += 1.0
    return dst, pows, cs, ns


def _build_matrix():
    dst, pows, cs, ns = _tables(_MAX_L)
    monos = sorted({tuple(p) for p in pows})
    midx = {m: i for i, m in enumerate(monos)}
    n_out = len(ns)
    mat = np.zeros((len(monos), n_out), dtype=np.float64)
    for d, p, c in zip(dst, pows, cs):
        mat[midx[tuple(p)], d] += c
    mat = mat * np.asarray(ns, dtype=np.float64)[None, :]
    return monos, mat.astype(np.float32)


_MONOS, _MAT = _build_matrix()
_N_MONO = len(_MONOS)
_N_OUT = _MAT.shape[1]

_G = 5
_BLOCK = 3200
_BG = _BLOCK // _G

_RHS = np.zeros((_N_MONO * _G, _N_OUT * _G), dtype=np.float32)
for _g in range(_G):
    for _m in range(_N_MONO):
        _RHS[_G * _m + _g, _g * _N_OUT:(_g + 1) * _N_OUT] = _MAT[_m]


def _body(x5_ref, m_ref, o_ref):
    x = x5_ref[0]
    y = x5_ref[1]
    z = x5_ref[2]
    xp = [None, x, x * x, None, None]
    yp = [None, y, y * y, None, None]
    zp = [None, z, z * z, None, None]
    xp[3], xp[4] = xp[2] * x, xp[2] * xp[2]
    yp[3], yp[4] = yp[2] * y, yp[2] * yp[2]
    zp[3], zp[4] = zp[2] * z, zp[2] * zp[2]
    rows = []
    for (a, b, c) in _MONOS:
        facs = []
        if a:
            facs.append(xp[a])
        if b:
            facs.append(yp[b])
        if c:
            facs.append(zp[c])
        if not facs:
            v = jnp.ones_like(x)
        else:
            v = facs[0]
            for f in facs[1:]:
                v = v * f
        rows.append(v)
    p = jnp.concatenate(rows, axis=0)
    o_ref[...] = jax.lax.dot_general(
        p, m_ref[...], (((0,), (0,)), ((), ())),
        preferred_element_type=jnp.float32)


def kernel(xyz):
    in_shape = xyz.shape
    x = xyz.reshape(-1, 3)
    n = x.shape[0]
    ng = n // _G
    x5 = jnp.transpose(x.reshape(ng, _G, 3), (2, 1, 0))
    rhs = jnp.asarray(_RHS)
    grid = n // _BLOCK
    out = pl.pallas_call(
        _body,
        grid=(grid,),
        in_specs=[
            pl.BlockSpec((3, _G, _BG), lambda i: (0, 0, i)),
            pl.BlockSpec(_RHS.shape, lambda i: (0, 0)),
        ],
        out_specs=pl.BlockSpec((_BG, _N_OUT * _G), lambda i: (i, 0)),
        out_shape=jax.ShapeDtypeStruct((ng, _N_OUT * _G), jnp.float32),
    )(x5, rhs)
    return out.reshape(*in_shape[:-1], _N_OUT)


# speedup vs baseline: 15.9249x; 15.9249x over previous
import numpy as np
from math import comb, factorial, floor

import jax
import jax.numpy as jnp
from jax.experimental import pallas as pl

_MAX_L = 4


def _tables(max_l):
    dst, pows, cs, ns = [], [], [], []
    for l in range(max_l + 1):
        for m in range(-l, l + 1):
            am = abs(m)
            n_lm = (1.0 / (2.0 ** am * factorial(l))) * np.sqrt(
                2.0 * factorial(l + am) * factorial(l - am) / (2.0 if m == 0 else 1.0))
            ns.append(n_lm)
            vm = 0.5 if m < 0 else 0.0
            vmax = floor(am / 2.0 - vm) + vm
            for t in range(0, (l - am) // 2 + 1):
                for u in range(0, t + 1):
                    v = vm
                    while v <= vmax + 1e-9:
                        c = ((-1.0) ** int(round(t + v - vm))) * (0.25 ** t) \
                            * comb(l, t) * comb(l - t, am + t) * comb(t, u) * comb(am, int(round(2 * v)))
                        dst.append(l * (l + 1) + m)
                        pows.append([int(round(2 * t + am - 2 * (u + v))),
                                     int(round(2 * (u + v))),
                                     int(l - 2 * t - am)])
                        cs.append(c)
                        v += 1.0
    return dst, pows, cs, ns


def _channel_terms():
    dst, pows, cs, ns = _tables(_MAX_L)
    n_out = len(ns)
    terms = {}
    for d, p, c in zip(dst, pows, cs):
        key = (d, tuple(p))
        terms[key] = terms.get(key, 0.0) + c
    chans = [[] for _ in range(n_out)]
    for (d, p), c in terms.items():
        chans[d].append((float(c) * float(ns[d]), p))
    return chans


_CHANS = _channel_terms()
_N_OUT = len(_CHANS)

_BLOCK = 16000


def _body(xt_ref, o_ref):
    x = xt_ref[0:1, :]
    y = xt_ref[1:2, :]
    z = xt_ref[2:3, :]
    xp = [None, x, x * x, None, None]
    yp = [None, y, y * y, None, None]
    zp = [None, z, z * z, None, None]
    xp[3], xp[4] = xp[2] * x, xp[2] * xp[2]
    yp[3], yp[4] = yp[2] * y, yp[2] * yp[2]
    zp[3], zp[4] = zp[2] * z, zp[2] * zp[2]
    pw = (xp, yp, zp)
    for j, terms in enumerate(_CHANS):
        acc = None
        for coef, (a, b, c) in terms:
            m = None
            for pwc, e in zip(pw, (a, b, c)):
                if e:
                    m = pwc[e] if m is None else m * pwc[e]
            if m is None:
                t = jnp.full_like(x, coef)
            else:
                t = m * coef
            acc = t if acc is None else acc + t
        o_ref[j:j + 1, :] = acc


def kernel(xyz):
    in_shape = xyz.shape
    x = xyz.reshape(-1, 3)
    n = x.shape[0]
    xt = x.T
    grid = n // _BLOCK
    out = pl.pallas_call(
        _body,
        grid=(grid,),
        in_specs=[pl.BlockSpec((3, _BLOCK), lambda i: (0, i))],
        out_specs=pl.BlockSpec((_N_OUT, _BLOCK), lambda i: (0, i)),
        out_shape=jax.ShapeDtypeStruct((_N_OUT, n), jnp.float32),
    )(xt)
    return out.T.reshape(*in_shape[:-1], _N_OUT)
